# baseline (speedup 1.0000x reference)
import functools

import jax
import jax.numpy as jnp
from jax import lax
from jax.experimental import pallas as pl
from jax.experimental.pallas import tpu as pltpu
from jax.experimental.pallas import tpu_sc as plsc

N_NODES = 10000
D_FEAT = 128
NUM_LAYERS = 3

NCORE = 2
NSUB = 16
NW = NCORE * NSUB
G = 128

N_ACC = 10112
ROWS_PER_SUB = N_ACC // NSUB

_mesh = plsc.VectorSubcoreMesh(core_axis_name="c", subcore_axis_name="s")


def _zero_fill(buf, nrows, width):
    z = jnp.zeros((16,), jnp.float32)

    @pl.loop(0, nrows)
    def _(r):
        @pl.loop(0, width // 16)
        def _(j):
            buf[r, pl.ds(j * 16, 16)] = z


def _copy_rows(src, dst, total_rows):
    off = 0
    while off < total_rows:
        n = min(128, total_rows - off)
        pltpu.sync_copy(src.at[pl.ds(0, n)], dst.at[pl.ds(off, n)])
        off += n


def _make_hist(ngroups):
    pw = ngroups // NW

    @functools.partial(
        pl.kernel,
        out_type=jax.ShapeDtypeStruct((NCORE, N_ACC, 16), jnp.float32),
        mesh=_mesh,
        scratch_types=[
            pltpu.VMEM((2, G), jnp.int32),
            pltpu.VMEM((G, 16), jnp.float32),
            pltpu.VMEM((128, 16), jnp.float32),
            pltpu.VMEM_SHARED((N_ACC, 16), jnp.float32),
        ],
    )
    def hist(col_hbm, out_hbm, idx_v, ones_v, zeros_v, deg_sh):
        c = lax.axis_index("c")
        s = lax.axis_index("s")

        one = jnp.ones((16,), jnp.float32)

        @pl.loop(0, G)
        def _(r):
            ones_v[r, :] = one

        _zero_fill(zeros_v, 128, 16)
        base = s * ROWS_PER_SUB
        off = 0
        while off < ROWS_PER_SUB:
            n = min(128, ROWS_PER_SUB - off)
            pltpu.sync_copy(zeros_v.at[pl.ds(0, n)],
                            deg_sh.at[pl.ds(base + off, n)])
            off += n
        plsc.subcore_barrier()

        g0 = c * (ngroups // NCORE) + s * pw

        @pl.loop(0, pw)
        def _(k):
            pltpu.sync_copy(col_hbm.at[g0 + k, 1], idx_v.at[0])
            pltpu.sync_copy(ones_v, deg_sh.at[idx_v.at[0]], add=True)

        plsc.subcore_barrier()
        pltpu.sync_copy(deg_sh.at[pl.ds(base, ROWS_PER_SUB)],
                        out_hbm.at[c].at[pl.ds(base, ROWS_PER_SUB)])

    return hist


def _make_layer(ngroups):
    pw = ngroups // NW
    assert pw % 4 == 0

    @functools.partial(
        pl.kernel,
        out_type=jax.ShapeDtypeStruct((NCORE, N_ACC, D_FEAT), jnp.float32),
        mesh=_mesh,
        scratch_types=[
            pltpu.VMEM((4, 2, G), jnp.int32),
            pltpu.VMEM((2, G, D_FEAT), jnp.float32),
            pltpu.VMEM_SHARED((N_ACC, D_FEAT), jnp.float32),
            pltpu.SemaphoreType.DMA,
            pltpu.SemaphoreType.DMA,
            pltpu.SemaphoreType.DMA,
            pltpu.SemaphoreType.DMA,
            pltpu.SemaphoreType.DMA,
        ],
    )
    def layer(g_hbm, rc_hbm, out_hbm, rcix, msgs, acc_sh,
              si0, si1, sg, ss0, ss1):
        c = lax.axis_index("c")
        s = lax.axis_index("s")
        sis = (si0, si1)
        sss = (ss0, ss1)

        _zero_fill(msgs.at[0], G, D_FEAT)
        base = s * ROWS_PER_SUB
        off = 0
        while off < ROWS_PER_SUB:
            n = min(128, ROWS_PER_SUB - off)
            pltpu.sync_copy(msgs.at[0].at[pl.ds(0, n)],
                            acc_sh.at[pl.ds(base + off, n)])
            off += n
        plsc.subcore_barrier()

        g0 = c * (ngroups // NCORE) + s * pw

        def wait_gather(buf):
            pltpu.make_async_copy(
                g_hbm.at[pl.ds(0, G)], msgs.at[buf], sg).wait()

        def wait_scatter(buf, par):
            pltpu.make_async_copy(
                msgs.at[buf], acc_sh.at[pl.ds(0, G)], sss[par]).wait()

        def wait_idx(slot, par):
            pltpu.make_async_copy(
                rc_hbm.at[g0], rcix.at[slot], sis[par]).wait()

        pltpu.async_copy(rc_hbm.at[g0], rcix.at[0], si0)
        pltpu.async_copy(rc_hbm.at[g0 + 1], rcix.at[1], si1)
        wait_idx(0, 0)
        pltpu.async_copy(g_hbm.at[rcix.at[0, 0]], msgs.at[0], sg)

        @pl.loop(0, pw // 4)
        def _(p):
            for q in range(4):
                k = p * 4 + q
                buf = q % 2
                wait_gather(buf)

                @pl.when(k > 0)
                def _():
                    wait_scatter(1 - buf, (q - 1) % 2)

                @pl.when(k + 2 < pw)
                def _():
                    pltpu.async_copy(rc_hbm.at[g0 + k + 2],
                                     rcix.at[(q + 2) % 4], sis[q % 2])

                @pl.when(k + 1 < pw)
                def _():
                    wait_idx((q + 1) % 4, (q + 1) % 2)
                    pltpu.async_copy(g_hbm.at[rcix.at[(q + 1) % 4, 0]],
                                     msgs.at[1 - buf], sg)

                pltpu.async_copy(msgs.at[buf], acc_sh.at[rcix.at[q, 1]],
                                 sss[q % 2], add=True)

        wait_scatter(1, (pw - 1) % 2)
        plsc.subcore_barrier()
        pltpu.sync_copy(acc_sh.at[pl.ds(base, ROWS_PER_SUB)],
                        out_hbm.at[c].at[pl.ds(base, ROWS_PER_SUB)])

    return layer


_BLK = 1000
_GRID = N_NODES // _BLK

_deg_spec = pl.BlockSpec((NCORE, _BLK, 16), lambda i: (0, i, 0))
_acc_spec = pl.BlockSpec((NCORE, _BLK, D_FEAT), lambda i: (0, i, 0))
_row_spec = pl.BlockSpec((_BLK, D_FEAT), lambda i: (i, 0))


def _dinv_of(deg_ref):
    deg = deg_ref[0, :, 0:1] + deg_ref[1, :, 0:1] + 1.0
    return lax.rsqrt(deg)


def _prescale_body(deg_ref, x_ref, g_ref):
    g_ref[...] = _dinv_of(deg_ref) * x_ref[...]


_prescale = pl.pallas_call(
    _prescale_body,
    grid=(_GRID,),
    in_specs=[_deg_spec, _row_spec],
    out_specs=_row_spec,
    out_shape=jax.ShapeDtypeStruct((N_NODES, D_FEAT), jnp.float32),
)


def _combine_mid_body(deg_ref, acc_ref, g_ref, sum_ref, gn_ref, sn_ref):
    dinv = _dinv_of(deg_ref)
    h = dinv * (acc_ref[0] + acc_ref[1] + g_ref[...])
    sn_ref[...] = sum_ref[...] + h
    gn_ref[...] = dinv * h


_combine_mid = pl.pallas_call(
    _combine_mid_body,
    grid=(_GRID,),
    in_specs=[_deg_spec, _acc_spec, _row_spec, _row_spec],
    out_specs=[_row_spec, _row_spec],
    out_shape=[jax.ShapeDtypeStruct((N_NODES, D_FEAT), jnp.float32),
               jax.ShapeDtypeStruct((N_NODES, D_FEAT), jnp.float32)],
)


def _combine_last_body(deg_ref, acc_ref, g_ref, sum_ref, out_ref):
    dinv = _dinv_of(deg_ref)
    h = dinv * (acc_ref[0] + acc_ref[1] + g_ref[...])
    out_ref[...] = (sum_ref[...] + h) * (1.0 / (NUM_LAYERS + 1))


_combine_last = pl.pallas_call(
    _combine_last_body,
    grid=(_GRID,),
    in_specs=[_deg_spec, _acc_spec, _row_spec, _row_spec],
    out_specs=_row_spec,
    out_shape=jax.ShapeDtypeStruct((N_NODES, D_FEAT), jnp.float32),
)


def kernel(x, edge_index):
    n_edges = edge_index.shape[1]
    pw = -(-n_edges // (G * NW))
    pw = -(-pw // 4) * 4
    ngroups = pw * NW
    pad = ngroups * G - n_edges

    row = edge_index[0]
    col = edge_index[1]
    if pad:
        row = jnp.concatenate([row, jnp.zeros((pad,), row.dtype)])
        col = jnp.concatenate([col, jnp.full((pad,), N_NODES, col.dtype)])
    rc2d = jnp.stack([row.reshape(ngroups, G), col.reshape(ngroups, G)],
                     axis=1)

    hist = _make_hist(ngroups)
    layer = _make_layer(ngroups)

    deg2 = hist(rc2d)
    g = _prescale(deg2, x)
    running = x
    for li in range(NUM_LAYERS):
        acc = layer(g, rc2d)
        if li < NUM_LAYERS - 1:
            g, running = _combine_mid(deg2, acc, g, running)
        else:
            out = _combine_last(deg2, acc, g, running)
    return out


# speedup vs baseline: 1.1052x; 1.1052x over previous
import functools

import jax
import jax.numpy as jnp
from jax import lax
from jax.experimental import pallas as pl
from jax.experimental.pallas import tpu as pltpu
from jax.experimental.pallas import tpu_sc as plsc

N_NODES = 10000
D_FEAT = 128
NUM_LAYERS = 3

NCORE = 2
NSUB = 16
NW = NCORE * NSUB
G = 128

PW_FAST = 124
PW_SLOW = 36

N_ACC = 10112
ROWS_PER_SUB = N_ACC // NSUB

_mesh = plsc.VectorSubcoreMesh(core_axis_name="c", subcore_axis_name="s",
                               num_cores=NCORE, num_subcores=NSUB)


def _zero_fill(buf, nrows, width):
    z = jnp.zeros((16,), jnp.float32)

    @pl.loop(0, nrows)
    def _(r):
        @pl.loop(0, width // 16)
        def _(j):
            buf[r, pl.ds(j * 16, 16)] = z


def _make_hist(ngroups):
    pw = ngroups // NW

    @functools.partial(
        pl.kernel,
        out_type=jax.ShapeDtypeStruct((NCORE, N_ACC, 16), jnp.float32),
        mesh=_mesh,
        scratch_types=[
            pltpu.VMEM((2, G), jnp.int32),
            pltpu.VMEM((G, 16), jnp.float32),
            pltpu.VMEM((128, 16), jnp.float32),
            pltpu.VMEM_SHARED((N_ACC, 16), jnp.float32),
        ],
    )
    def hist(col_hbm, out_hbm, idx_v, ones_v, zeros_v, deg_sh):
        c = lax.axis_index("c")
        s = lax.axis_index("s")

        one = jnp.ones((16,), jnp.float32)

        @pl.loop(0, G)
        def _(r):
            ones_v[r, :] = one

        _zero_fill(zeros_v, 128, 16)
        base = s * ROWS_PER_SUB
        off = 0
        while off < ROWS_PER_SUB:
            n = min(128, ROWS_PER_SUB - off)
            pltpu.sync_copy(zeros_v.at[pl.ds(0, n)],
                            deg_sh.at[pl.ds(base + off, n)])
            off += n
        plsc.subcore_barrier()

        g0 = c * (ngroups // NCORE) + s * pw

        @pl.loop(0, pw)
        def _(k):
            pltpu.sync_copy(col_hbm.at[g0 + k, 1], idx_v.at[0])
            pltpu.sync_copy(ones_v, deg_sh.at[idx_v.at[0]], add=True)

        plsc.subcore_barrier()
        pltpu.sync_copy(deg_sh.at[pl.ds(base, ROWS_PER_SUB)],
                        out_hbm.at[c].at[pl.ds(base, ROWS_PER_SUB)])

    return hist


def _make_layer(ngroups):
    assert (PW_FAST + PW_SLOW) * NSUB == ngroups
    assert PW_FAST % 4 == 0 and PW_SLOW % 4 == 0

    @functools.partial(
        pl.kernel,
        out_type=jax.ShapeDtypeStruct((NCORE, N_ACC, D_FEAT), jnp.float32),
        mesh=_mesh,
        scratch_types=[
            pltpu.VMEM((4, 2, G), jnp.int32),
            pltpu.VMEM((2, G, D_FEAT), jnp.float32),
            pltpu.VMEM_SHARED((N_ACC, D_FEAT), jnp.float32),
            pltpu.SemaphoreType.DMA,
            pltpu.SemaphoreType.DMA,
            pltpu.SemaphoreType.DMA,
            pltpu.SemaphoreType.DMA,
            pltpu.SemaphoreType.DMA,
        ],
    )
    def layer(g_hbm, rc_hbm, out_hbm, rcix, msgs, acc_sh,
              si0, si1, sg, ss0, ss1):
        c = lax.axis_index("c")
        s = lax.axis_index("s")
        sis = (si0, si1)
        sss = (ss0, ss1)

        _zero_fill(msgs.at[0], G, D_FEAT)
        base = s * ROWS_PER_SUB
        off = 0
        while off < ROWS_PER_SUB:
            n = min(128, ROWS_PER_SUB - off)
            pltpu.sync_copy(msgs.at[0].at[pl.ds(0, n)],
                            acc_sh.at[pl.ds(base + off, n)])
            off += n
        plsc.subcore_barrier()

        def wait_gather(buf):
            pltpu.make_async_copy(
                g_hbm.at[pl.ds(0, G)], msgs.at[buf], sg).wait()

        def wait_scatter(buf, par):
            pltpu.make_async_copy(
                msgs.at[buf], acc_sh.at[pl.ds(0, G)], sss[par]).wait()

        def wait_idx(slot, par):
            pltpu.make_async_copy(
                rc_hbm.at[0], rcix.at[slot], sis[par]).wait()

        g0 = jnp.where(c == 0, s * PW_FAST, NSUB * PW_FAST + s * PW_SLOW)
        pw = jnp.where(c == 0, PW_FAST, PW_SLOW)

        pltpu.async_copy(rc_hbm.at[g0], rcix.at[0], si0)
        pltpu.async_copy(rc_hbm.at[g0 + 1], rcix.at[1], si1)
        wait_idx(0, 0)
        pltpu.async_copy(g_hbm.at[rcix.at[0, 0]], msgs.at[0], sg)

        @pl.loop(0, pw // 4)
        def _(p):
            for q in range(4):
                k = p * 4 + q
                buf = q % 2
                wait_gather(buf)

                @pl.when(k > 0)
                def _():
                    wait_scatter(1 - buf, (q - 1) % 2)

                @pl.when(k + 2 < pw)
                def _():
                    pltpu.async_copy(rc_hbm.at[g0 + k + 2],
                                     rcix.at[(q + 2) % 4], sis[q % 2])

                @pl.when(k + 1 < pw)
                def _():
                    wait_idx((q + 1) % 4, (q + 1) % 2)
                    pltpu.async_copy(g_hbm.at[rcix.at[(q + 1) % 4, 0]],
                                     msgs.at[1 - buf], sg)

                pltpu.async_copy(msgs.at[buf], acc_sh.at[rcix.at[q, 1]],
                                 sss[q % 2], add=True)

        wait_scatter(1, 1)
        plsc.subcore_barrier()
        pltpu.sync_copy(acc_sh.at[pl.ds(base, ROWS_PER_SUB)],
                        out_hbm.at[c].at[pl.ds(base, ROWS_PER_SUB)])

    return layer


_BLK = 1000
_GRID = N_NODES // _BLK

_deg_spec = pl.BlockSpec((NCORE, _BLK, 16), lambda i: (0, i, 0))
_acc_spec = pl.BlockSpec((NCORE, _BLK, D_FEAT), lambda i: (0, i, 0))
_row_spec = pl.BlockSpec((_BLK, D_FEAT), lambda i: (i, 0))

_row_shape = jax.ShapeDtypeStruct((N_NODES, D_FEAT), jnp.float32)


def _dinv_of(deg_ref):
    deg = deg_ref[0, :, 0:1] + deg_ref[1, :, 0:1] + 1.0
    return lax.rsqrt(deg)


def _prescale_body(deg_ref, x_ref, g_ref):
    g_ref[...] = _dinv_of(deg_ref) * x_ref[...]


_prescale = pl.pallas_call(
    _prescale_body,
    grid=(_GRID,),
    in_specs=[_deg_spec, _row_spec],
    out_specs=_row_spec,
    out_shape=_row_shape,
)


def _combine_mid_body(deg_ref, acc_ref, g_ref, sum_ref, gn_ref, sn_ref):
    dinv = _dinv_of(deg_ref)
    h = dinv * (acc_ref[0] + acc_ref[1] + g_ref[...])
    sn_ref[...] = sum_ref[...] + h
    gn_ref[...] = dinv * h


_combine_mid = pl.pallas_call(
    _combine_mid_body,
    grid=(_GRID,),
    in_specs=[_deg_spec, _acc_spec, _row_spec, _row_spec],
    out_specs=[_row_spec, _row_spec],
    out_shape=[_row_shape, _row_shape],
)


def _combine_last_body(deg_ref, acc_ref, g_ref, sum_ref, out_ref):
    dinv = _dinv_of(deg_ref)
    h = dinv * (acc_ref[0] + acc_ref[1] + g_ref[...])
    out_ref[...] = (sum_ref[...] + h) * (1.0 / (NUM_LAYERS + 1))


_combine_last = pl.pallas_call(
    _combine_last_body,
    grid=(_GRID,),
    in_specs=[_deg_spec, _acc_spec, _row_spec, _row_spec],
    out_specs=_row_spec,
    out_shape=_row_shape,
)


def kernel(x, edge_index):
    n_edges = edge_index.shape[1]
    pw = -(-n_edges // (G * NW))
    pw = -(-pw // 4) * 4
    ngroups = pw * NW
    pad = ngroups * G - n_edges

    row = edge_index[0]
    col = edge_index[1]
    if pad:
        row = jnp.concatenate([row, jnp.zeros((pad,), row.dtype)])
        col = jnp.concatenate([col, jnp.full((pad,), N_NODES, col.dtype)])
    rc2d = jnp.stack([row.reshape(ngroups, G), col.reshape(ngroups, G)],
                     axis=1)

    hist = _make_hist(ngroups)
    layer = _make_layer(ngroups)

    deg2 = hist(rc2d)
    g = _prescale(deg2, x)
    running = x
    for li in range(NUM_LAYERS):
        acc = layer(g, rc2d)
        if li < NUM_LAYERS - 1:
            g, running = _combine_mid(deg2, acc, g, running)
        else:
            out = _combine_last(deg2, acc, g, running)
    return out
